# Initial kernel scaffold; baseline (speedup 1.0000x reference)
import functools

import jax
import jax.numpy as jnp
from jax import lax
from jax.experimental import pallas as pl
from jax.experimental.pallas import tpu as pltpu
from jax.experimental.pallas import tpu_sc as plsc

N_NODES = 10000
N_EDGES = 320000
EMBED = 128
HEADS = 8
DH = EMBED // HEADS

NC = 2
NS = 16
NW = NC * NS
CH = 128
EPW = 10240
EPAD = EPW * NW
NCHUNK = EPW // CH
STRIPE = N_NODES // NS

_mesh = plsc.VectorSubcoreMesh(
    core_axis_name="c", subcore_axis_name="s", num_cores=NC, num_subcores=NS)


def _qkv_body(x_ref, w_ref, o_ref):
    o_ref[...] = jnp.dot(x_ref[...], w_ref[...],
                         preferred_element_type=jnp.float32)


def _tc_qkv(embeds, w):
    return pl.pallas_call(
        _qkv_body,
        grid=(10,),
        in_specs=[pl.BlockSpec((N_NODES // 10, EMBED), lambda i: (i, 0)),
                  pl.BlockSpec((EMBED, 3 * EMBED), lambda i: (0, 0))],
        out_specs=pl.BlockSpec((N_NODES // 10, 3 * EMBED), lambda i: (i, 0)),
        out_shape=jax.ShapeDtypeStruct((N_NODES, 3 * EMBED), jnp.float32),
    )(embeds, w)


@functools.partial(
    pl.kernel,
    out_type=[jax.ShapeDtypeStruct((EPAD, EMBED), jnp.float32)] * 3,
    mesh=_mesh,
    scratch_types=[
        pltpu.VMEM((CH,), jnp.int32),
        pltpu.VMEM((CH,), jnp.int32),
        pltpu.VMEM((CH, EMBED), jnp.float32),
        pltpu.VMEM((CH, EMBED), jnp.float32),
        pltpu.VMEM((CH, EMBED), jnp.float32),
        pltpu.SemaphoreType.DMA,
        pltpu.SemaphoreType.DMA,
        pltpu.SemaphoreType.DMA,
    ],
)
def _sc_gather(q_hbm, k_hbm, v_hbm, rows_hbm, cols_hbm,
               qr_hbm, kc_hbm, vc_hbm,
               ridx, cidx, qbuf, kbuf, vbuf, s0, s1, s2):
    wid = lax.axis_index("s") * NC + lax.axis_index("c")
    base = wid * EPW

    @pl.loop(0, NCHUNK)
    def _(ci):
        off = base + ci * CH
        pltpu.sync_copy(rows_hbm.at[pl.ds(off, CH)], ridx)
        pltpu.sync_copy(cols_hbm.at[pl.ds(off, CH)], cidx)
        g0 = pltpu.async_copy(q_hbm.at[ridx], qbuf, s0)
        g1 = pltpu.async_copy(k_hbm.at[cidx], kbuf, s1)
        g2 = pltpu.async_copy(v_hbm.at[cidx], vbuf, s2)
        g0.wait()
        w0 = pltpu.async_copy(qbuf, qr_hbm.at[pl.ds(off, CH)], s0)
        g1.wait()
        w1 = pltpu.async_copy(kbuf, kc_hbm.at[pl.ds(off, CH)], s1)
        g2.wait()
        w2 = pltpu.async_copy(vbuf, vc_hbm.at[pl.ds(off, CH)], s2)
        w0.wait()
        w1.wait()
        w2.wait()


def _edge_body(qr_ref, kc_ref, vc_ref, m_ref, mt_ref, ex_ref, res_ref):
    i = pl.program_id(0)
    blk = qr_ref.shape[0]
    att = jnp.dot(qr_ref[...] * kc_ref[...], m_ref[...],
                  preferred_element_type=jnp.float32)
    att = jnp.exp(jnp.clip(att, -10.0, 10.0))
    eid = i * blk + lax.broadcasted_iota(jnp.int32, att.shape, 0)
    att = jnp.where(eid < N_EDGES, att, 0.0)
    ex_ref[...] = att
    res_ref[...] = vc_ref[...] * jnp.dot(att, mt_ref[...],
                                         preferred_element_type=jnp.float32)


def _tc_edge(qr, kc, vc, m, mt):
    blk = 4096
    nblk = EPAD // blk
    return pl.pallas_call(
        _edge_body,
        grid=(nblk,),
        in_specs=[pl.BlockSpec((blk, EMBED), lambda i: (i, 0)),
                  pl.BlockSpec((blk, EMBED), lambda i: (i, 0)),
                  pl.BlockSpec((blk, EMBED), lambda i: (i, 0)),
                  pl.BlockSpec((EMBED, 16), lambda i: (0, 0)),
                  pl.BlockSpec((16, EMBED), lambda i: (0, 0))],
        out_specs=[pl.BlockSpec((blk, 16), lambda i: (i, 0)),
                   pl.BlockSpec((blk, EMBED), lambda i: (i, 0))],
        out_shape=[jax.ShapeDtypeStruct((EPAD, 16), jnp.float32),
                   jax.ShapeDtypeStruct((EPAD, EMBED), jnp.float32)],
    )(qr, kc, vc, m, mt)


@functools.partial(
    pl.kernel,
    out_type=[jax.ShapeDtypeStruct((NC, N_NODES, EMBED), jnp.float32),
              jax.ShapeDtypeStruct((NC, N_NODES, 16), jnp.float32)],
    mesh=_mesh,
    scratch_types=[
        pltpu.VMEM_SHARED((N_NODES, EMBED), jnp.float32),
        pltpu.VMEM_SHARED((N_NODES, 16), jnp.float32),
        pltpu.VMEM((CH,), jnp.int32),
        pltpu.VMEM((CH, EMBED), jnp.float32),
        pltpu.VMEM((CH, 16), jnp.float32),
        pltpu.SemaphoreType.DMA,
        pltpu.SemaphoreType.DMA,
    ],
)
def _sc_scatter(rows_hbm, res_hbm, ex_hbm, z128_hbm, z16_hbm,
                sp_hbm, np_hbm,
                s_sp, n_sp, ridx, rbuf, ebuf, s0, s1):
    c = lax.axis_index("c")
    s = lax.axis_index("s")
    wid = s * NC + c
    r0 = s * STRIPE
    pltpu.sync_copy(z128_hbm.at[pl.ds(r0, STRIPE)], s_sp.at[pl.ds(r0, STRIPE)])
    pltpu.sync_copy(z16_hbm.at[pl.ds(r0, STRIPE)], n_sp.at[pl.ds(r0, STRIPE)])
    plsc.subcore_barrier()

    base = wid * EPW

    @pl.loop(0, NCHUNK)
    def _(ci):
        off = base + ci * CH
        pltpu.sync_copy(rows_hbm.at[pl.ds(off, CH)], ridx)
        a = pltpu.async_copy(res_hbm.at[pl.ds(off, CH)], rbuf, s0)
        b = pltpu.async_copy(ex_hbm.at[pl.ds(off, CH)], ebuf, s1)
        a.wait()
        pltpu.sync_copy(rbuf, s_sp.at[ridx], add=True)
        b.wait()
        pltpu.sync_copy(ebuf, n_sp.at[ridx], add=True)

    plsc.subcore_barrier()
    pltpu.sync_copy(s_sp.at[pl.ds(r0, STRIPE)], sp_hbm.at[c, pl.ds(r0, STRIPE)])
    pltpu.sync_copy(n_sp.at[pl.ds(r0, STRIPE)], np_hbm.at[c, pl.ds(r0, STRIPE)])


def _fin_body(sp_ref, np_ref, mt_ref, o_ref):
    ssum = sp_ref[0] + sp_ref[1]
    nsum = np_ref[0] + np_ref[1]
    denom = jnp.dot(nsum, mt_ref[...],
                    preferred_element_type=jnp.float32) + 1e-8
    o_ref[...] = ssum / denom


def _tc_finalize(sp, npart, mt):
    blk = N_NODES // 10
    return pl.pallas_call(
        _fin_body,
        grid=(10,),
        in_specs=[pl.BlockSpec((NC, blk, EMBED), lambda i: (0, i, 0)),
                  pl.BlockSpec((NC, blk, 16), lambda i: (0, i, 0)),
                  pl.BlockSpec((16, EMBED), lambda i: (0, 0))],
        out_specs=pl.BlockSpec((blk, EMBED), lambda i: (i, 0)),
        out_shape=jax.ShapeDtypeStruct((N_NODES, EMBED), jnp.float32),
    )(sp, npart, mt)


def kernel(embeds, qTrans, kTrans, vTrans, edge_index):
    rows = edge_index[0, :].astype(jnp.int32)
    cols = edge_index[1, :].astype(jnp.int32)
    pad = EPAD - N_EDGES
    rows_p = jnp.concatenate([rows, jnp.zeros((pad,), jnp.int32)])
    cols_p = jnp.concatenate([cols, jnp.zeros((pad,), jnp.int32)])

    w = jnp.concatenate([qTrans, kTrans, vTrans], axis=1)
    qkv = _tc_qkv(embeds, w)
    q = qkv[:, :EMBED]
    k = qkv[:, EMBED:2 * EMBED]
    v = qkv[:, 2 * EMBED:]

    qr, kc, vc = _sc_gather(q, k, v, rows_p, cols_p)

    d_iota = jnp.arange(EMBED, dtype=jnp.int32)
    h_iota = jnp.arange(16, dtype=jnp.int32)
    m = (d_iota[:, None] // DH == h_iota[None, :]).astype(jnp.float32)
    mt = m.T

    ex, res = _tc_edge(qr, kc, vc, m, mt)

    z128 = jnp.zeros((N_NODES, EMBED), jnp.float32)
    z16 = jnp.zeros((N_NODES, 16), jnp.float32)
    sp, npart = _sc_scatter(rows_p, res, ex, z128, z16)

    return _tc_finalize(sp, npart, mt)


# speedup vs baseline: 2.4560x; 2.4560x over previous
import functools

import jax
import jax.numpy as jnp
from jax import lax
from jax.experimental import pallas as pl
from jax.experimental.pallas import tpu as pltpu
from jax.experimental.pallas import tpu_sc as plsc

N_NODES = 10000
N_EDGES = 320000
EMBED = 128
HEADS = 8
DH = EMBED // HEADS

NC = 2
NS = 16
NW = NC * NS
CH = 128
EPW = 10240
EPAD = EPW * NW
NCHUNK = EPW // CH
STRIPE = 1000

def _qkv_body(x_ref, w_ref, o_ref):
    o_ref[...] = jnp.dot(x_ref[...], w_ref[...],
                         preferred_element_type=jnp.float32)


def _tc_qkv(embeds, w):
    return pl.pallas_call(
        _qkv_body,
        grid=(10,),
        in_specs=[pl.BlockSpec((N_NODES // 10, EMBED), lambda i: (i, 0)),
                  pl.BlockSpec((EMBED, 3 * EMBED), lambda i: (0, 0))],
        out_specs=pl.BlockSpec((N_NODES // 10, 3 * EMBED), lambda i: (i, 0)),
        out_shape=jax.ShapeDtypeStruct((N_NODES, 3 * EMBED), jnp.float32),
    )(embeds, w)


@functools.cache
def _make_sc_gather():
  mesh = plsc.VectorSubcoreMesh(
      core_axis_name="c", subcore_axis_name="s",
      num_cores=NC, num_subcores=NS)

  @functools.partial(
    pl.kernel,
    out_type=[jax.ShapeDtypeStruct((EPAD, EMBED), jnp.float32)] * 3,
    mesh=mesh,
    scratch_types=[
        pltpu.VMEM((CH,), jnp.int32),
        pltpu.VMEM((CH,), jnp.int32),
        pltpu.VMEM((CH, EMBED), jnp.float32),
        pltpu.VMEM((CH, EMBED), jnp.float32),
        pltpu.VMEM((CH, EMBED), jnp.float32),
        pltpu.SemaphoreType.DMA,
        pltpu.SemaphoreType.DMA,
        pltpu.SemaphoreType.DMA,
    ],
  )
  def sc_gather(q_hbm, k_hbm, v_hbm, rows_hbm, cols_hbm,
                qr_hbm, kc_hbm, vc_hbm,
                ridx, cidx, qbuf, kbuf, vbuf, s0, s1, s2):
    wid = lax.axis_index("s") * NC + lax.axis_index("c")
    base = wid * EPW

    @pl.loop(0, NCHUNK)
    def _(ci):
        off = base + ci * CH
        pltpu.sync_copy(rows_hbm.at[pl.ds(off, CH)], ridx)
        pltpu.sync_copy(cols_hbm.at[pl.ds(off, CH)], cidx)
        g0 = pltpu.async_copy(q_hbm.at[ridx], qbuf, s0)
        g1 = pltpu.async_copy(k_hbm.at[cidx], kbuf, s1)
        g2 = pltpu.async_copy(v_hbm.at[cidx], vbuf, s2)
        g0.wait()
        w0 = pltpu.async_copy(qbuf, qr_hbm.at[pl.ds(off, CH)], s0)
        g1.wait()
        w1 = pltpu.async_copy(kbuf, kc_hbm.at[pl.ds(off, CH)], s1)
        g2.wait()
        w2 = pltpu.async_copy(vbuf, vc_hbm.at[pl.ds(off, CH)], s2)
        w0.wait()
        w1.wait()
        w2.wait()

  return sc_gather


def _edge_body(qr_ref, kc_ref, vc_ref, m_ref, mt_ref, res_ref, attw_ref):
    i = pl.program_id(0)
    blk = qr_ref.shape[0]
    att = jnp.dot(qr_ref[...] * kc_ref[...], m_ref[...],
                  preferred_element_type=jnp.float32)
    att = jnp.exp(jnp.clip(att, -10.0, 10.0))
    eid = i * blk + lax.broadcasted_iota(jnp.int32, att.shape, 0)
    att = jnp.where(eid < N_EDGES, att, 0.0)
    attw = jnp.dot(att, mt_ref[...], preferred_element_type=jnp.float32)
    res_ref[...] = vc_ref[...] * attw
    attw_ref[...] = attw


def _tc_edge(qr, kc, vc, m, mt):
    blk = 4096
    nblk = EPAD // blk
    return pl.pallas_call(
        _edge_body,
        grid=(nblk,),
        in_specs=[pl.BlockSpec((blk, EMBED), lambda i: (i, 0)),
                  pl.BlockSpec((blk, EMBED), lambda i: (i, 0)),
                  pl.BlockSpec((blk, EMBED), lambda i: (i, 0)),
                  pl.BlockSpec((EMBED, 16), lambda i: (0, 0)),
                  pl.BlockSpec((16, EMBED), lambda i: (0, 0))],
        out_specs=[pl.BlockSpec((blk, EMBED), lambda i: (i, 0)),
                   pl.BlockSpec((blk, EMBED), lambda i: (i, 0))],
        out_shape=[jax.ShapeDtypeStruct((EPAD, EMBED), jnp.float32),
                   jax.ShapeDtypeStruct((EPAD, EMBED), jnp.float32)],
    )(qr, kc, vc, m, mt)


@functools.cache
def _make_sc_scatter():
  mesh = plsc.VectorSubcoreMesh(
      core_axis_name="c", subcore_axis_name="s",
      num_cores=NC, num_subcores=NS)

  @functools.partial(
    pl.kernel,
    out_type=[jax.ShapeDtypeStruct((NC, N_NODES, EMBED), jnp.float32),
              jax.ShapeDtypeStruct((NC, N_NODES, EMBED), jnp.float32)],
    mesh=mesh,
    scratch_types=[
        pltpu.VMEM_SHARED((N_NODES, EMBED), jnp.float32),
        pltpu.VMEM((1, CH), jnp.int32),
        pltpu.VMEM((CH, EMBED), jnp.float32),
    ],
  )
  def sc_scatter(rows_hbm, res_hbm, attw_hbm, z_hbm,
                 acc_hbm, nacc_hbm,
                 a_sp, ridx2, rbuf):
    c = lax.axis_index("c")
    s = lax.axis_index("s")
    wid = s * NC + c
    r0 = s * STRIPE
    base = wid * EPW

    def one_pass(src_hbm, dst_hbm):
        @pl.when(s < N_NODES // STRIPE)
        def _():
            pltpu.sync_copy(z_hbm.at[pl.ds(r0, STRIPE)],
                            a_sp.at[pl.ds(r0, STRIPE)])

        plsc.subcore_barrier()

        @pl.loop(0, NCHUNK)
        def _(ci):
            off = base + ci * CH
            pltpu.sync_copy(rows_hbm.at[pl.ds(off, CH)], ridx2.at[0])
            pltpu.sync_copy(src_hbm.at[pl.ds(off, CH)], rbuf)
            pltpu.sync_copy(rbuf, a_sp.at[ridx2.at[0]], add=True)

        plsc.subcore_barrier()

        @pl.when(s < N_NODES // STRIPE)
        def _():
            pltpu.sync_copy(a_sp.at[pl.ds(r0, STRIPE)],
                            dst_hbm.at[c, pl.ds(r0, STRIPE)])

    one_pass(res_hbm, acc_hbm)
    one_pass(attw_hbm, nacc_hbm)

  return sc_scatter


def _fin_body(acc_ref, nacc_ref, o_ref):
    ssum = acc_ref[0] + acc_ref[1]
    denom = nacc_ref[0] + nacc_ref[1] + 1e-8
    o_ref[...] = ssum / denom


def _tc_finalize(acc, nacc):
    blk = N_NODES // 10
    return pl.pallas_call(
        _fin_body,
        grid=(10,),
        in_specs=[pl.BlockSpec((NC, blk, EMBED), lambda i: (0, i, 0)),
                  pl.BlockSpec((NC, blk, EMBED), lambda i: (0, i, 0))],
        out_specs=pl.BlockSpec((blk, EMBED), lambda i: (i, 0)),
        out_shape=jax.ShapeDtypeStruct((N_NODES, EMBED), jnp.float32),
    )(acc, nacc)


def kernel(embeds, qTrans, kTrans, vTrans, edge_index):
    rows = edge_index[0, :].astype(jnp.int32)
    cols = edge_index[1, :].astype(jnp.int32)
    pad = EPAD - N_EDGES
    rows_p = jnp.concatenate([rows, jnp.zeros((pad,), jnp.int32)])
    cols_p = jnp.concatenate([cols, jnp.zeros((pad,), jnp.int32)])

    w = jnp.concatenate([qTrans, kTrans, vTrans], axis=1)
    qkv = _tc_qkv(embeds, w)
    q = qkv[:, :EMBED]
    k = qkv[:, EMBED:2 * EMBED]
    v = qkv[:, 2 * EMBED:]

    qr, kc, vc = _make_sc_gather()(q, k, v, rows_p, cols_p)

    d_iota = jnp.arange(EMBED, dtype=jnp.int32)
    h_iota = jnp.arange(16, dtype=jnp.int32)
    m = (d_iota[:, None] // DH == h_iota[None, :]).astype(jnp.float32)
    mt = m.T

    res, attw = _tc_edge(qr, kc, vc, m, mt)

    z = jnp.zeros((N_NODES, EMBED), jnp.float32)
    acc, nacc = _make_sc_scatter()(rows_p, res, attw, z)

    return _tc_finalize(acc, nacc)


# speedup vs baseline: 2.7887x; 1.1355x over previous
import functools

import jax
import jax.numpy as jnp
from jax import lax
from jax.experimental import pallas as pl
from jax.experimental.pallas import tpu as pltpu
from jax.experimental.pallas import tpu_sc as plsc

N_NODES = 10000
N_EDGES = 320000
EMBED = 128
HEADS = 8
DH = EMBED // HEADS

NC = 2
NS = 16
NW = NC * NS
CH = 128
EPW = 10240
EPAD = EPW * NW
NCHUNK = EPW // CH
STRIPE = 1000

def _qkv_body(x_ref, w_ref, o_ref):
    o_ref[...] = jnp.dot(x_ref[...], w_ref[...],
                         preferred_element_type=jnp.float32)


def _tc_qkv(embeds, w):
    return pl.pallas_call(
        _qkv_body,
        grid=(10,),
        in_specs=[pl.BlockSpec((N_NODES // 10, EMBED), lambda i: (i, 0)),
                  pl.BlockSpec((EMBED, 3 * EMBED), lambda i: (0, 0))],
        out_specs=pl.BlockSpec((N_NODES // 10, 3 * EMBED), lambda i: (i, 0)),
        out_shape=jax.ShapeDtypeStruct((N_NODES, 3 * EMBED), jnp.float32),
    )(embeds, w)


@functools.cache
def _make_sc_gather():
  mesh = plsc.VectorSubcoreMesh(
      core_axis_name="c", subcore_axis_name="s",
      num_cores=NC, num_subcores=NS)

  @functools.partial(
    pl.kernel,
    out_type=[jax.ShapeDtypeStruct((EPAD, EMBED), jnp.float32),
              jax.ShapeDtypeStruct((EPAD, 2 * EMBED), jnp.float32)],
    mesh=mesh,
    scratch_types=[
        pltpu.VMEM((2, 1, CH), jnp.int32),
        pltpu.VMEM((2, 1, CH), jnp.int32),
        pltpu.VMEM((2, CH, EMBED), jnp.float32),
        pltpu.VMEM((2, CH, 2 * EMBED), jnp.float32),
        pltpu.SemaphoreType.DMA,
        pltpu.SemaphoreType.DMA,
        pltpu.SemaphoreType.DMA,
        pltpu.SemaphoreType.DMA,
        pltpu.SemaphoreType.DMA,
        pltpu.SemaphoreType.DMA,
    ],
  )
  def sc_gather(q_hbm, kv_hbm, rows_hbm, cols_hbm,
                qr_hbm, kvc_hbm,
                ridx, cidx, qbuf, kvbuf, si0, si1, sq0, sq1, sk0, sk1):
    wid = lax.axis_index("s") * NC + lax.axis_index("c")
    base = wid * EPW
    sqs = (sq0, sq1)
    sks = (sk0, sk1)
    sis = (si0, si1)

    @pl.loop(0, NCHUNK // 2)
    def _(it):
        c0 = it * 2
        offs = (base + c0 * CH, base + c0 * CH + CH)
        ii = []
        for b in (0, 1):
            ii.append(pltpu.async_copy(
                rows_hbm.at[pl.ds(offs[b], CH)], ridx.at[b, 0], sis[b]))
            ii.append(pltpu.async_copy(
                cols_hbm.at[pl.ds(offs[b], CH)], cidx.at[b, 0], sis[b]))
        gg = []
        for b in (0, 1):
            ii[2 * b].wait()
            ii[2 * b + 1].wait()
            gg.append(pltpu.async_copy(
                q_hbm.at[ridx.at[b, 0]], qbuf.at[b], sqs[b]))
            gg.append(pltpu.async_copy(
                kv_hbm.at[cidx.at[b, 0]], kvbuf.at[b], sks[b]))
        ww = []
        for b in (0, 1):
            gg[2 * b].wait()
            ww.append(pltpu.async_copy(
                qbuf.at[b], qr_hbm.at[pl.ds(offs[b], CH)], sqs[b]))
            gg[2 * b + 1].wait()
            ww.append(pltpu.async_copy(
                kvbuf.at[b], kvc_hbm.at[pl.ds(offs[b], CH)], sks[b]))
        for w in ww:
            w.wait()

  return sc_gather


def _edge_body(qr_ref, kvc_ref, m_ref, mt_ref, res_ref, attw_ref):
    i = pl.program_id(0)
    blk = qr_ref.shape[0]
    kc = kvc_ref[:, :EMBED]
    vc = kvc_ref[:, EMBED:]
    att = jnp.dot(qr_ref[...] * kc, m_ref[...],
                  preferred_element_type=jnp.float32)
    att = jnp.exp(jnp.clip(att, -10.0, 10.0))
    eid = i * blk + lax.broadcasted_iota(jnp.int32, att.shape, 0)
    att = jnp.where(eid < N_EDGES, att, 0.0)
    attw = jnp.dot(att, mt_ref[...], preferred_element_type=jnp.float32)
    res_ref[...] = vc * attw
    attw_ref[...] = attw


def _tc_edge(qr, kvc, m, mt):
    blk = 4096
    nblk = EPAD // blk
    return pl.pallas_call(
        _edge_body,
        grid=(nblk,),
        in_specs=[pl.BlockSpec((blk, EMBED), lambda i: (i, 0)),
                  pl.BlockSpec((blk, 2 * EMBED), lambda i: (i, 0)),
                  pl.BlockSpec((EMBED, 16), lambda i: (0, 0)),
                  pl.BlockSpec((16, EMBED), lambda i: (0, 0))],
        out_specs=[pl.BlockSpec((blk, EMBED), lambda i: (i, 0)),
                   pl.BlockSpec((blk, EMBED), lambda i: (i, 0))],
        out_shape=[jax.ShapeDtypeStruct((EPAD, EMBED), jnp.float32),
                   jax.ShapeDtypeStruct((EPAD, EMBED), jnp.float32)],
    )(qr, kvc, m, mt)


@functools.cache
def _make_sc_scatter():
  mesh = plsc.VectorSubcoreMesh(
      core_axis_name="c", subcore_axis_name="s",
      num_cores=NC, num_subcores=NS)

  @functools.partial(
    pl.kernel,
    out_type=[jax.ShapeDtypeStruct((NC, N_NODES, EMBED), jnp.float32),
              jax.ShapeDtypeStruct((NC, N_NODES, EMBED), jnp.float32)],
    mesh=mesh,
    scratch_types=[
        pltpu.VMEM_SHARED((N_NODES, EMBED), jnp.float32),
        pltpu.VMEM((2, 1, CH), jnp.int32),
        pltpu.VMEM((2, CH, EMBED), jnp.float32),
        pltpu.SemaphoreType.DMA,
        pltpu.SemaphoreType.DMA,
        pltpu.SemaphoreType.DMA,
        pltpu.SemaphoreType.DMA,
    ],
  )
  def sc_scatter(rows_hbm, res_hbm, attw_hbm, z_hbm,
                 acc_hbm, nacc_hbm,
                 a_sp, ridx, rbuf, si0, si1, sd0, sd1):
    c = lax.axis_index("c")
    s = lax.axis_index("s")
    wid = s * NC + c
    r0 = s * STRIPE
    base = wid * EPW
    sis = (si0, si1)
    sds = (sd0, sd1)

    def one_pass(src_hbm, dst_hbm):
        @pl.when(s < N_NODES // STRIPE)
        def _():
            pltpu.sync_copy(z_hbm.at[pl.ds(r0, STRIPE)],
                            a_sp.at[pl.ds(r0, STRIPE)])

        plsc.subcore_barrier()

        @pl.loop(0, NCHUNK // 2)
        def _(it):
            c0 = it * 2
            offs = (base + c0 * CH, base + c0 * CH + CH)
            ii = []
            ll = []
            for b in (0, 1):
                ii.append(pltpu.async_copy(
                    rows_hbm.at[pl.ds(offs[b], CH)], ridx.at[b, 0], sis[b]))
                ll.append(pltpu.async_copy(
                    src_hbm.at[pl.ds(offs[b], CH)], rbuf.at[b], sds[b]))
            ss = []
            for b in (0, 1):
                ii[b].wait()
                ll[b].wait()
                ss.append(pltpu.async_copy(
                    rbuf.at[b], a_sp.at[ridx.at[b, 0]], sds[b], add=True))
            for x in ss:
                x.wait()

        plsc.subcore_barrier()

        @pl.when(s < N_NODES // STRIPE)
        def _():
            pltpu.sync_copy(a_sp.at[pl.ds(r0, STRIPE)],
                            dst_hbm.at[c, pl.ds(r0, STRIPE)])

    one_pass(res_hbm, acc_hbm)
    one_pass(attw_hbm, nacc_hbm)

  return sc_scatter


def _fin_body(acc_ref, nacc_ref, o_ref):
    ssum = acc_ref[0] + acc_ref[1]
    denom = nacc_ref[0] + nacc_ref[1] + 1e-8
    o_ref[...] = ssum / denom


def _tc_finalize(acc, nacc):
    blk = N_NODES // 10
    return pl.pallas_call(
        _fin_body,
        grid=(10,),
        in_specs=[pl.BlockSpec((NC, blk, EMBED), lambda i: (0, i, 0)),
                  pl.BlockSpec((NC, blk, EMBED), lambda i: (0, i, 0))],
        out_specs=pl.BlockSpec((blk, EMBED), lambda i: (i, 0)),
        out_shape=jax.ShapeDtypeStruct((N_NODES, EMBED), jnp.float32),
    )(acc, nacc)


def kernel(embeds, qTrans, kTrans, vTrans, edge_index):
    rows = edge_index[0, :].astype(jnp.int32)
    cols = edge_index[1, :].astype(jnp.int32)
    pad = EPAD - N_EDGES
    rows_p = jnp.concatenate([rows, jnp.zeros((pad,), jnp.int32)])
    cols_p = jnp.concatenate([cols, jnp.zeros((pad,), jnp.int32)])

    w = jnp.concatenate([qTrans, kTrans, vTrans], axis=1)
    qkv = _tc_qkv(embeds, w)
    q = qkv[:, :EMBED]
    kv = qkv[:, EMBED:]

    qr, kvc = _make_sc_gather()(q, kv, rows_p, cols_p)

    d_iota = jnp.arange(EMBED, dtype=jnp.int32)
    h_iota = jnp.arange(16, dtype=jnp.int32)
    m = (d_iota[:, None] // DH == h_iota[None, :]).astype(jnp.float32)
    mt = m.T

    res, attw = _tc_edge(qr, kvc, m, mt)

    z = jnp.zeros((N_NODES, EMBED), jnp.float32)
    acc, nacc = _make_sc_scatter()(rows_p, res, attw, z)

    return _tc_finalize(acc, nacc)


# speedup vs baseline: 3.1309x; 1.1227x over previous
import functools

import jax
import jax.numpy as jnp
from jax import lax
from jax.experimental import pallas as pl
from jax.experimental.pallas import tpu as pltpu
from jax.experimental.pallas import tpu_sc as plsc

N_NODES = 10000
N_EDGES = 320000
EMBED = 128
HEADS = 8
DH = EMBED // HEADS

NC = 2
NS = 16
NW = NC * NS
CH = 128
EPAD = 327680
HALF = EPAD // 2
EPWH = HALF // NW
NCHUNKH = EPWH // CH
STRIPE = 1000


def _qkv_body(x_ref, w_ref, o_ref):
    o_ref[...] = jnp.dot(x_ref[...], w_ref[...],
                         preferred_element_type=jnp.float32)


def _tc_qkv(embeds, w):
    return pl.pallas_call(
        _qkv_body,
        grid=(10,),
        in_specs=[pl.BlockSpec((N_NODES // 10, EMBED), lambda i: (i, 0)),
                  pl.BlockSpec((EMBED, 3 * EMBED), lambda i: (0, 0))],
        out_specs=pl.BlockSpec((N_NODES // 10, 3 * EMBED), lambda i: (i, 0)),
        out_shape=jax.ShapeDtypeStruct((N_NODES, 3 * EMBED), jnp.float32),
    )(embeds, w)


@functools.cache
def _make_sc_gather(tag):
  mesh = plsc.VectorSubcoreMesh(
      core_axis_name="c", subcore_axis_name="s",
      num_cores=NC, num_subcores=NS)

  @functools.partial(
    pl.kernel,
    out_type=[jax.ShapeDtypeStruct((HALF, EMBED), jnp.float32),
              jax.ShapeDtypeStruct((HALF, 2 * EMBED), jnp.float32)],
    mesh=mesh,
    scratch_types=[
        pltpu.VMEM((2, 1, CH), jnp.int32),
        pltpu.VMEM((2, 1, CH), jnp.int32),
        pltpu.VMEM((2, CH, EMBED), jnp.float32),
        pltpu.VMEM((2, CH, 2 * EMBED), jnp.float32),
        pltpu.SemaphoreType.DMA,
        pltpu.SemaphoreType.DMA,
        pltpu.SemaphoreType.DMA,
        pltpu.SemaphoreType.DMA,
        pltpu.SemaphoreType.DMA,
        pltpu.SemaphoreType.DMA,
    ],
  )
  def sc_gather(q_hbm, kv_hbm, rows_hbm, cols_hbm,
                qr_hbm, kvc_hbm,
                ridx, cidx, qbuf, kvbuf, si0, si1, sq0, sq1, sk0, sk1):
    wid = lax.axis_index("s") * NC + lax.axis_index("c")
    base = wid * EPWH
    sqs = (sq0, sq1)
    sks = (sk0, sk1)
    sis = (si0, si1)

    @pl.loop(0, NCHUNKH // 2)
    def _(it):
        c0 = it * 2
        offs = (base + c0 * CH, base + c0 * CH + CH)
        ii = []
        for b in (0, 1):
            ii.append(pltpu.async_copy(
                rows_hbm.at[pl.ds(offs[b], CH)], ridx.at[b, 0], sis[b]))
            ii.append(pltpu.async_copy(
                cols_hbm.at[pl.ds(offs[b], CH)], cidx.at[b, 0], sis[b]))
        gg = []
        for b in (0, 1):
            ii[2 * b].wait()
            ii[2 * b + 1].wait()
            gg.append(pltpu.async_copy(
                q_hbm.at[ridx.at[b, 0]], qbuf.at[b], sqs[b]))
            gg.append(pltpu.async_copy(
                kv_hbm.at[cidx.at[b, 0]], kvbuf.at[b], sks[b]))
        ww = []
        for b in (0, 1):
            gg[2 * b].wait()
            ww.append(pltpu.async_copy(
                qbuf.at[b], qr_hbm.at[pl.ds(offs[b], CH)], sqs[b]))
            gg[2 * b + 1].wait()
            ww.append(pltpu.async_copy(
                kvbuf.at[b], kvc_hbm.at[pl.ds(offs[b], CH)], sks[b]))
        for w in ww:
            w.wait()

  return sc_gather


def _make_edge_body(h):
    def _edge_body(qr_ref, kvc_ref, m_ref, mt_ref, res_ref, attw_ref):
        i = pl.program_id(0)
        blk = qr_ref.shape[0]
        kc = kvc_ref[:, :EMBED]
        vc = kvc_ref[:, EMBED:]
        att = jnp.dot(qr_ref[...] * kc, m_ref[...],
                      preferred_element_type=jnp.float32)
        att = jnp.exp(jnp.clip(att, -10.0, 10.0))
        eid = h * HALF + i * blk + lax.broadcasted_iota(jnp.int32, att.shape, 0)
        att = jnp.where(eid < N_EDGES, att, 0.0)
        attw = jnp.dot(att, mt_ref[...], preferred_element_type=jnp.float32)
        res_ref[...] = vc * attw
        attw_ref[...] = attw
    return _edge_body


def _tc_edge(qr, kvc, m, mt, h):
    blk = 4096
    nblk = HALF // blk
    return pl.pallas_call(
        _make_edge_body(h),
        grid=(nblk,),
        in_specs=[pl.BlockSpec((blk, EMBED), lambda i: (i, 0)),
                  pl.BlockSpec((blk, 2 * EMBED), lambda i: (i, 0)),
                  pl.BlockSpec((EMBED, 16), lambda i: (0, 0)),
                  pl.BlockSpec((16, EMBED), lambda i: (0, 0))],
        out_specs=[pl.BlockSpec((blk, EMBED), lambda i: (i, 0)),
                   pl.BlockSpec((blk, EMBED), lambda i: (i, 0))],
        out_shape=[jax.ShapeDtypeStruct((HALF, EMBED), jnp.float32),
                   jax.ShapeDtypeStruct((HALF, EMBED), jnp.float32)],
    )(qr, kvc, m, mt)


@functools.cache
def _make_sc_scatter(tag):
  mesh = plsc.VectorSubcoreMesh(
      core_axis_name="c", subcore_axis_name="s",
      num_cores=NC, num_subcores=NS)

  @functools.partial(
    pl.kernel,
    out_type=[jax.ShapeDtypeStruct((NC, N_NODES, EMBED), jnp.float32),
              jax.ShapeDtypeStruct((NC, N_NODES, EMBED), jnp.float32)],
    mesh=mesh,
    scratch_types=[
        pltpu.VMEM_SHARED((N_NODES, EMBED), jnp.float32),
        pltpu.VMEM((2, 1, CH), jnp.int32),
        pltpu.VMEM((2, CH, EMBED), jnp.float32),
        pltpu.SemaphoreType.DMA,
        pltpu.SemaphoreType.DMA,
        pltpu.SemaphoreType.DMA,
        pltpu.SemaphoreType.DMA,
    ],
  )
  def sc_scatter(rows_hbm, res_hbm, attw_hbm, z_hbm,
                 acc_hbm, nacc_hbm,
                 a_sp, ridx, rbuf, si0, si1, sd0, sd1):
    c = lax.axis_index("c")
    s = lax.axis_index("s")
    wid = s * NC + c
    r0 = s * STRIPE
    base = wid * EPWH
    sis = (si0, si1)
    sds = (sd0, sd1)

    def one_pass(src_hbm, dst_hbm):
        @pl.when(s < N_NODES // STRIPE)
        def _():
            pltpu.sync_copy(z_hbm.at[pl.ds(r0, STRIPE)],
                            a_sp.at[pl.ds(r0, STRIPE)])

        plsc.subcore_barrier()

        @pl.loop(0, NCHUNKH // 2)
        def _(it):
            c0 = it * 2
            offs = (base + c0 * CH, base + c0 * CH + CH)
            ii = []
            ll = []
            for b in (0, 1):
                ii.append(pltpu.async_copy(
                    rows_hbm.at[pl.ds(offs[b], CH)], ridx.at[b, 0], sis[b]))
                ll.append(pltpu.async_copy(
                    src_hbm.at[pl.ds(offs[b], CH)], rbuf.at[b], sds[b]))
            ss = []
            for b in (0, 1):
                ii[b].wait()
                ll[b].wait()
                ss.append(pltpu.async_copy(
                    rbuf.at[b], a_sp.at[ridx.at[b, 0]], sds[b], add=True))
            for x in ss:
                x.wait()

        plsc.subcore_barrier()

        @pl.when(s < N_NODES // STRIPE)
        def _():
            pltpu.sync_copy(a_sp.at[pl.ds(r0, STRIPE)],
                            dst_hbm.at[c, pl.ds(r0, STRIPE)])

    one_pass(res_hbm, acc_hbm)
    one_pass(attw_hbm, nacc_hbm)

  return sc_scatter


def _fin_body(aa_ref, na_ref, ab_ref, nb_ref, o_ref):
    ssum = aa_ref[0] + aa_ref[1] + ab_ref[0] + ab_ref[1]
    denom = na_ref[0] + na_ref[1] + nb_ref[0] + nb_ref[1] + 1e-8
    o_ref[...] = ssum / denom


def _tc_finalize(acc_a, nacc_a, acc_b, nacc_b):
    blk = N_NODES // 10
    spec = pl.BlockSpec((NC, blk, EMBED), lambda i: (0, i, 0))
    return pl.pallas_call(
        _fin_body,
        grid=(10,),
        in_specs=[spec, spec, spec, spec],
        out_specs=pl.BlockSpec((blk, EMBED), lambda i: (i, 0)),
        out_shape=jax.ShapeDtypeStruct((N_NODES, EMBED), jnp.float32),
    )(acc_a, nacc_a, acc_b, nacc_b)


def kernel(embeds, qTrans, kTrans, vTrans, edge_index):
    rows = edge_index[0, :].astype(jnp.int32)
    cols = edge_index[1, :].astype(jnp.int32)
    pad = EPAD - N_EDGES
    rows_p = jnp.concatenate([rows, jnp.zeros((pad,), jnp.int32)])
    cols_p = jnp.concatenate([cols, jnp.zeros((pad,), jnp.int32)])

    w = jnp.concatenate([qTrans, kTrans, vTrans], axis=1)
    qkv = _tc_qkv(embeds, w)
    q = qkv[:, :EMBED]
    kv = qkv[:, EMBED:]

    d_iota = jnp.arange(EMBED, dtype=jnp.int32)
    h_iota = jnp.arange(16, dtype=jnp.int32)
    m = (d_iota[:, None] // DH == h_iota[None, :]).astype(jnp.float32)
    mt = m.T

    z = jnp.zeros((N_NODES, EMBED), jnp.float32)

    halves = []
    for h in (0, 1):
        rows_h = lax.slice(rows_p, (h * HALF,), ((h + 1) * HALF,))
        cols_h = lax.slice(cols_p, (h * HALF,), ((h + 1) * HALF,))
        qr, kvc = _make_sc_gather(h)(q, kv, rows_h, cols_h)
        res, attw = _tc_edge(qr, kvc, m, mt, h)
        acc, nacc = _make_sc_scatter(h)(rows_h, res, attw, z)
        halves.append((acc, nacc))

    (acc_a, nacc_a), (acc_b, nacc_b) = halves
    return _tc_finalize(acc_a, nacc_a, acc_b, nacc_b)
